# Initial kernel scaffold; baseline (speedup 1.0000x reference)
import jax
import jax.numpy as jnp
from jax.experimental import pallas as pl


def kernel(xyz1, xyz2, points1, points2, W0, b0, g0, beta0, W1, b1, g1, beta1):
    raise NotImplementedError("write your pallas kernel here")


# speedup vs baseline: 25.5001x; 25.5001x over previous
import functools

import jax
import jax.numpy as jnp
import numpy as np
from jax import lax
from jax.experimental import pallas as pl
from jax.experimental.pallas import tpu as pltpu
from jax.experimental.pallas import tpu_sc as plsc

_B, _N1, _N2, _C1, _C2 = 4, 8192, 2048, 64, 128
_NP = _B * _N1
_TILE = 256
_MT = 2048
_NW = 32
_PPW = _NP // _NW
_CH = 128
_NCHUNK = _PPW // _CH


def _knn_body(x1_ref, x2t_ref, i0, i1, i2, w0, w1, w2):
    b = pl.program_id(0)
    x1 = x1_ref[0]
    x2 = x2t_ref[0]
    c0, c1, c2 = x1[:, 0:1], x1[:, 1:2], x1[:, 2:3]
    r0, r1, r2 = x2[0:1, :], x2[1:2, :], x2[2:3, :]
    dot = lax.dot_general(x1, x2, (((1,), (0,)), ((), ())),
                          preferred_element_type=jnp.float32)
    x1sq = (c0 * c0 + c1 * c1) + c2 * c2
    x2sq = (r0 * r0 + r1 * r1) + r2 * r2
    d = -2.0 * dot + x1sq + x2sq
    iota = lax.broadcasted_iota(jnp.int32, (_TILE, _N2), 1)
    idxs, dists = [], []
    for k in range(3):
        m = jnp.min(d, axis=1, keepdims=True)
        am = jnp.min(jnp.where(d == m, iota, _N2), axis=1, keepdims=True)
        dists.append(m)
        idxs.append(am)
        if k < 2:
            d = jnp.where(iota == am, jnp.float32(np.inf), d)
    ws = [1.0 / jnp.maximum(x, 1e-10) for x in dists]
    norm = jnp.maximum((ws[0] + ws[1]) + ws[2], 1e-8)
    base = b * _N2
    for ref, am in zip((i0, i1, i2), idxs):
        ref[0] = am + base
    for ref, wv in zip((w0, w1, w2), ws):
        ref[0] = wv / norm


def _knn(xyz1, xyz2t):
    idx_spec = pl.BlockSpec((1, _TILE, 1), lambda b, t: (b, t, 0))
    shapes = ([jax.ShapeDtypeStruct((_B, _N1, 1), jnp.int32)] * 3
              + [jax.ShapeDtypeStruct((_B, _N1, 1), jnp.float32)] * 3)
    return pl.pallas_call(
        _knn_body,
        grid=(_B, _N1 // _TILE),
        in_specs=[pl.BlockSpec((1, _TILE, 3), lambda b, t: (b, t, 0)),
                  pl.BlockSpec((1, 3, _N2), lambda b, t: (b, 0, 0))],
        out_specs=[idx_spec] * 6,
        out_shape=shapes,
    )(xyz1, xyz2t)


def _sc_interp(idx0, idx1, idx2, w0, w1, w2, table):
    mesh = plsc.VectorSubcoreMesh(core_axis_name="c", subcore_axis_name="s")

    @functools.partial(
        pl.kernel,
        out_type=jax.ShapeDtypeStruct((_NP, _C2), jnp.float32),
        mesh=mesh,
        scratch_types=[
            pltpu.VMEM((_CH,), jnp.int32),
            pltpu.VMEM((_CH,), jnp.int32),
            pltpu.VMEM((_CH,), jnp.int32),
            pltpu.VMEM((_CH + 16,), jnp.float32),
            pltpu.VMEM((_CH + 16,), jnp.float32),
            pltpu.VMEM((_CH + 16,), jnp.float32),
            pltpu.VMEM((_CH, _C2), jnp.float32),
            pltpu.VMEM((_CH, _C2), jnp.float32),
            pltpu.VMEM((_CH, _C2), jnp.float32),
            pltpu.VMEM((_CH, _C2), jnp.float32),
            pltpu.SemaphoreType.DMA,
        ],
    )
    def kfn(i0h, i1h, i2h, w0h, w1h, w2h, th, outh,
            i0v, i1v, i2v, w0v, w1v, w2v, r0v, r1v, r2v, ov, sem):
        wid = lax.axis_index("c") * 16 + lax.axis_index("s")
        for ci in range(_NCHUNK):
            base = wid * _PPW + ci * _CH
            sl = pl.ds(base, _CH)
            pltpu.sync_copy(i0h.at[sl], i0v)
            pltpu.sync_copy(i1h.at[sl], i1v)
            pltpu.sync_copy(i2h.at[sl], i2v)
            pltpu.sync_copy(w0h.at[sl], w0v.at[pl.ds(0, _CH)])
            pltpu.sync_copy(w1h.at[sl], w1v.at[pl.ds(0, _CH)])
            pltpu.sync_copy(w2h.at[sl], w2v.at[pl.ds(0, _CH)])
            d0 = pltpu.async_copy(th.at[i0v], r0v, sem)
            d1 = pltpu.async_copy(th.at[i1v], r1v, sem)
            d2 = pltpu.async_copy(th.at[i2v], r2v, sem)
            d0.wait()
            d1.wait()
            d2.wait()

            def body(q, carry):
                wv0 = w0v[pl.ds(q * 16, 16)]
                wv1 = w1v[pl.ds(q * 16, 16)]
                wv2 = w2v[pl.ds(q * 16, 16)]
                for j in range(16):
                    p = q * 16 + j
                    wb0 = jnp.full((16,), wv0[j], jnp.float32)
                    wb1 = jnp.full((16,), wv1[j], jnp.float32)
                    wb2 = jnp.full((16,), wv2[j], jnp.float32)
                    for g in range(_C2 // 16):
                        s = pl.ds(g * 16, 16)
                        ov[p, s] = (wb0 * r0v[p, s] + wb1 * r1v[p, s]
                                    + wb2 * r2v[p, s])
                return carry

            lax.fori_loop(0, _CH // 16, body, 0)
            pltpu.sync_copy(ov, outh.at[sl])

    return kfn(idx0, idx1, idx2, w0, w1, w2, table)


def _dot(a, b):
    return jnp.dot(a, b, preferred_element_type=jnp.float32,
                   precision=lax.Precision.HIGHEST)


def _acc_stats(i, y, st_ref):
    @pl.when(i == 0)
    def _():
        st_ref[...] = jnp.zeros_like(st_ref)
    st_ref[0:1, :] = st_ref[0:1, :] + jnp.sum(y, axis=0, keepdims=True)
    st_ref[1:2, :] = st_ref[1:2, :] + jnp.sum(y * y, axis=0, keepdims=True)


def _bn(y, st_ref, g_ref, be_ref):
    mean = st_ref[0:1, :] * (1.0 / _NP)
    q = st_ref[1:2, :] * (1.0 / _NP)
    rstd = lax.rsqrt((q - mean * mean) + 1e-5)
    return jnp.maximum((y - mean) * rstd * g_ref[...] + be_ref[...], 0.0)


def _mlp0_body(p1_ref, it_ref, wa_ref, wb_ref, b0_ref, y_ref, st_ref):
    i = pl.program_id(0)
    y = _dot(p1_ref[...], wa_ref[...]) + _dot(it_ref[...], wb_ref[...])
    y = y + b0_ref[...]
    y_ref[...] = y
    _acc_stats(i, y, st_ref)


def _mlp1_body(y0_ref, st_ref, g_ref, be_ref, w_ref, b_ref, y1_ref, st1_ref):
    i = pl.program_id(0)
    h = _bn(y0_ref[...], st_ref, g_ref, be_ref)
    y = _dot(h, w_ref[...]) + b_ref[...]
    y1_ref[...] = y
    _acc_stats(i, y, st1_ref)


def _mlp2_body(y1_ref, st_ref, g_ref, be_ref, o_ref):
    o_ref[...] = _bn(y1_ref[...], st_ref, g_ref, be_ref)


def _row_spec(c):
    return pl.BlockSpec((_MT, c), lambda i: (i, 0))


def _full_spec(r, c):
    return pl.BlockSpec((r, c), lambda i: (0, 0))


def _mlp0(p1, interp, waT, wbT, b0):
    return pl.pallas_call(
        _mlp0_body,
        grid=(_NP // _MT,),
        in_specs=[_row_spec(_C1), _row_spec(_C2), _full_spec(_C1, 128),
                  _full_spec(_C2, 128), _full_spec(1, 128)],
        out_specs=[_row_spec(128), _full_spec(8, 128)],
        out_shape=[jax.ShapeDtypeStruct((_NP, 128), jnp.float32),
                   jax.ShapeDtypeStruct((8, 128), jnp.float32)],
    )(p1, interp, waT, wbT, b0)


def _mlp1(y0, st0, g0, beta0, w1T, b1):
    return pl.pallas_call(
        _mlp1_body,
        grid=(_NP // _MT,),
        in_specs=[_row_spec(128), _full_spec(8, 128), _full_spec(1, 128),
                  _full_spec(1, 128), _full_spec(128, 64), _full_spec(1, 64)],
        out_specs=[_row_spec(64), _full_spec(8, 64)],
        out_shape=[jax.ShapeDtypeStruct((_NP, 64), jnp.float32),
                   jax.ShapeDtypeStruct((8, 64), jnp.float32)],
    )(y0, st0, g0, beta0, w1T, b1)


def _mlp2(y1, st1, g1, beta1):
    return pl.pallas_call(
        _mlp2_body,
        grid=(_NP // _MT,),
        in_specs=[_row_spec(64), _full_spec(8, 64), _full_spec(1, 64),
                  _full_spec(1, 64)],
        out_specs=_row_spec(64),
        out_shape=jax.ShapeDtypeStruct((_NP, 64), jnp.float32),
    )(y1, st1, g1, beta1)


def kernel(xyz1, xyz2, points1, points2, W0, b0, g0, beta0,
           W1, b1, g1, beta1):
    xyz2t = jnp.transpose(xyz2, (0, 2, 1))
    i0, i1, i2, w0, w1, w2 = _knn(xyz1, xyz2t)
    table = points2.reshape(_B * _N2, _C2)
    interp = _sc_interp(i0.reshape(_NP), i1.reshape(_NP), i2.reshape(_NP),
                        w0.reshape(_NP), w1.reshape(_NP), w2.reshape(_NP),
                        table)
    p1 = points1.reshape(_NP, _C1)
    waT = W0[:, :_C1].T
    wbT = W0[:, _C1:].T
    y0, st0 = _mlp0(p1, interp, waT, wbT, b0.reshape(1, 128))
    y1, st1 = _mlp1(y0, st0, g0.reshape(1, 128), beta0.reshape(1, 128),
                    W1.T, b1.reshape(1, 64))
    out = _mlp2(y1, st1, g1.reshape(1, 64), beta1.reshape(1, 64))
    return out.reshape(_B, _N1, _C1)


# speedup vs baseline: 31.0624x; 1.2181x over previous
import functools

import jax
import jax.numpy as jnp
import numpy as np
from jax import lax
from jax.experimental import pallas as pl
from jax.experimental.pallas import tpu as pltpu
from jax.experimental.pallas import tpu_sc as plsc

_B, _N1, _N2, _C1, _C2 = 4, 8192, 2048, 64, 128
_NP = _B * _N1
_TILE = 512
_NT = _N1 // _TILE
_MT = 2048
_NW = 32
_PPW = _NP // _NW
_CH = 128
_TPW = _PPW // _TILE
_HPT = _TILE // _CH


def _knn_body(x1_ref, x2t_ref, i0, w0):
    b = pl.program_id(0)
    x1 = x1_ref[0]
    x2 = x2t_ref[0]
    c0, c1, c2 = x1[:, 0:1], x1[:, 1:2], x1[:, 2:3]
    r0, r1, r2 = x2[0:1, :], x2[1:2, :], x2[2:3, :]
    dot = lax.dot_general(x1, x2, (((1,), (0,)), ((), ())),
                          preferred_element_type=jnp.float32)
    x1sq = (c0 * c0 + c1 * c1) + c2 * c2
    x2sq = (r0 * r0 + r1 * r1) + r2 * r2
    d = -2.0 * dot + x1sq + x2sq
    iota = lax.broadcasted_iota(jnp.int32, (_TILE, _N2), 1)
    idxs, dists = [], []
    for k in range(3):
        m = jnp.min(d, axis=1, keepdims=True)
        am = jnp.min(jnp.where(d == m, iota, _N2), axis=1, keepdims=True)
        dists.append(m)
        idxs.append(am)
        if k < 2:
            d = jnp.where(iota == am, jnp.float32(np.inf), d)
    ws = [1.0 / jnp.maximum(x, 1e-10) for x in dists]
    norm = jnp.maximum((ws[0] + ws[1]) + ws[2], 1e-8)
    base = b * _N2
    zi = jnp.zeros((_TILE, 1), jnp.int32)
    zf = jnp.zeros((_TILE, 1), jnp.float32)
    idx_mat = jnp.concatenate(
        [idxs[0] + base, idxs[1] + base, idxs[2] + base, zi, zi, zi, zi, zi],
        axis=1)
    w_mat = jnp.concatenate(
        [ws[0] / norm, ws[1] / norm, ws[2] / norm, zf, zf, zf, zf, zf],
        axis=1)
    i0[0, 0] = jnp.transpose(idx_mat, (1, 0))
    w0[0, 0] = jnp.transpose(w_mat, (1, 0))


def _knn(xyz1, xyz2t):
    out_spec = pl.BlockSpec((1, 1, 8, _TILE), lambda b, t: (b, t, 0, 0))
    shapes = [jax.ShapeDtypeStruct((_B, _NT, 8, _TILE), jnp.int32),
              jax.ShapeDtypeStruct((_B, _NT, 8, _TILE), jnp.float32)]
    return pl.pallas_call(
        _knn_body,
        grid=(_B, _NT),
        in_specs=[pl.BlockSpec((1, _TILE, 3), lambda b, t: (b, t, 0)),
                  pl.BlockSpec((1, 3, _N2), lambda b, t: (b, 0, 0))],
        out_specs=[out_spec] * 2,
        out_shape=shapes,
    )(xyz1, xyz2t)


def _sc_interp(idx, wgt, table):
    mesh = plsc.VectorSubcoreMesh(core_axis_name="c", subcore_axis_name="s")

    @functools.partial(
        pl.kernel,
        out_type=jax.ShapeDtypeStruct((_NP, _C2), jnp.float32),
        mesh=mesh,
        scratch_types=[
            pltpu.VMEM((8, _TILE), jnp.int32),
            pltpu.VMEM((8, _TILE), jnp.float32),
            pltpu.VMEM((_CH, _C2), jnp.float32),
            pltpu.VMEM((_CH, _C2), jnp.float32),
            pltpu.VMEM((_CH, _C2), jnp.float32),
            pltpu.VMEM((_CH, _C2), jnp.float32),
            pltpu.SemaphoreType.DMA,
        ],
    )
    def kfn(ih, wh, th, outh, iv, wv, r0v, r1v, r2v, ov, sem):
        wid = lax.axis_index("c") * 16 + lax.axis_index("s")
        for j in range(_TPW):
            g = wid * _TPW + j
            b = g // _NT
            tt = g % _NT
            pltpu.sync_copy(ih.at[b, tt], iv)
            pltpu.sync_copy(wh.at[b, tt], wv)
            for h in range(_HPT):
                base = b * _N1 + tt * _TILE + h * _CH
                hs = pl.ds(h * _CH, _CH)
                d0 = pltpu.async_copy(th.at[iv.at[0, hs]], r0v, sem)
                d1 = pltpu.async_copy(th.at[iv.at[1, hs]], r1v, sem)
                d2 = pltpu.async_copy(th.at[iv.at[2, hs]], r2v, sem)
                d0.wait()
                d1.wait()
                d2.wait()

                def body(q, carry):
                    qs = pl.ds(h * _CH + q * 16, 16)
                    wv0 = wv[0, qs]
                    wv1 = wv[1, qs]
                    wv2 = wv[2, qs]
                    for jj in range(16):
                        p = q * 16 + jj
                        wb0 = jnp.full((16,), wv0[jj], jnp.float32)
                        wb1 = jnp.full((16,), wv1[jj], jnp.float32)
                        wb2 = jnp.full((16,), wv2[jj], jnp.float32)
                        for gg in range(_C2 // 16):
                            s = pl.ds(gg * 16, 16)
                            ov[p, s] = (wb0 * r0v[p, s] + wb1 * r1v[p, s]
                                        + wb2 * r2v[p, s])
                    return carry

                lax.fori_loop(0, _CH // 16, body, 0)
                pltpu.sync_copy(ov, outh.at[pl.ds(base, _CH)])

    return kfn(idx, wgt, table)


def _dot(a, b):
    return jnp.dot(a, b, preferred_element_type=jnp.float32,
                   precision=lax.Precision.HIGHEST)


def _acc_stats(i, y, st_ref):
    @pl.when(i == 0)
    def _():
        st_ref[...] = jnp.zeros_like(st_ref)
    st_ref[0:1, :] = st_ref[0:1, :] + jnp.sum(y, axis=0, keepdims=True)
    st_ref[1:2, :] = st_ref[1:2, :] + jnp.sum(y * y, axis=0, keepdims=True)


def _bn(y, st_ref, g_ref, be_ref):
    mean = st_ref[0:1, :] * (1.0 / _NP)
    q = st_ref[1:2, :] * (1.0 / _NP)
    rstd = lax.rsqrt((q - mean * mean) + 1e-5)
    return jnp.maximum((y - mean) * rstd * g_ref[...] + be_ref[...], 0.0)


def _mlp0_body(p1_ref, it_ref, wa_ref, wb_ref, b0_ref, y_ref, st_ref):
    i = pl.program_id(0)
    y = _dot(p1_ref[...], wa_ref[...]) + _dot(it_ref[...], wb_ref[...])
    y = y + b0_ref[...]
    y_ref[...] = y
    _acc_stats(i, y, st_ref)


def _mlp1_body(y0_ref, st_ref, g_ref, be_ref, w_ref, b_ref, y1_ref, st1_ref):
    i = pl.program_id(0)
    h = _bn(y0_ref[...], st_ref, g_ref, be_ref)
    y = _dot(h, w_ref[...]) + b_ref[...]
    y1_ref[...] = y
    _acc_stats(i, y, st1_ref)


def _mlp2_body(y1_ref, st_ref, g_ref, be_ref, o_ref):
    o_ref[...] = _bn(y1_ref[...], st_ref, g_ref, be_ref)


def _row_spec(c):
    return pl.BlockSpec((_MT, c), lambda i: (i, 0))


def _full_spec(r, c):
    return pl.BlockSpec((r, c), lambda i: (0, 0))


def _mlp0(p1, interp, waT, wbT, b0):
    return pl.pallas_call(
        _mlp0_body,
        grid=(_NP // _MT,),
        in_specs=[_row_spec(_C1), _row_spec(_C2), _full_spec(_C1, 128),
                  _full_spec(_C2, 128), _full_spec(1, 128)],
        out_specs=[_row_spec(128), _full_spec(8, 128)],
        out_shape=[jax.ShapeDtypeStruct((_NP, 128), jnp.float32),
                   jax.ShapeDtypeStruct((8, 128), jnp.float32)],
    )(p1, interp, waT, wbT, b0)


def _mlp1(y0, st0, g0, beta0, w1T, b1):
    return pl.pallas_call(
        _mlp1_body,
        grid=(_NP // _MT,),
        in_specs=[_row_spec(128), _full_spec(8, 128), _full_spec(1, 128),
                  _full_spec(1, 128), _full_spec(128, 64), _full_spec(1, 64)],
        out_specs=[_row_spec(64), _full_spec(8, 64)],
        out_shape=[jax.ShapeDtypeStruct((_NP, 64), jnp.float32),
                   jax.ShapeDtypeStruct((8, 64), jnp.float32)],
    )(y0, st0, g0, beta0, w1T, b1)


def _mlp2(y1, st1, g1, beta1):
    return pl.pallas_call(
        _mlp2_body,
        grid=(_NP // _MT,),
        in_specs=[_row_spec(64), _full_spec(8, 64), _full_spec(1, 64),
                  _full_spec(1, 64)],
        out_specs=_row_spec(64),
        out_shape=jax.ShapeDtypeStruct((_NP, 64), jnp.float32),
    )(y1, st1, g1, beta1)


def kernel(xyz1, xyz2, points1, points2, W0, b0, g0, beta0,
           W1, b1, g1, beta1):
    xyz2t = jnp.transpose(xyz2, (0, 2, 1))
    idx, wgt = _knn(xyz1, xyz2t)
    table = points2.reshape(_B * _N2, _C2)
    interp = _sc_interp(idx, wgt, table)
    p1 = points1.reshape(_NP, _C1)
    waT = W0[:, :_C1].T
    wbT = W0[:, _C1:].T
    y0, st0 = _mlp0(p1, interp, waT, wbT, b0.reshape(1, 128))
    y1, st1 = _mlp1(y0, st0, g0.reshape(1, 128), beta0.reshape(1, 128),
                    W1.T, b1.reshape(1, 64))
    out = _mlp2(y1, st1, g1.reshape(1, 64), beta1.reshape(1, 64))
    return out.reshape(_B, _N1, _C1)


# speedup vs baseline: 31.9168x; 1.0275x over previous
import functools

import jax
import jax.numpy as jnp
import numpy as np
from jax import lax
from jax.experimental import pallas as pl
from jax.experimental.pallas import tpu as pltpu
from jax.experimental.pallas import tpu_sc as plsc

_B, _N1, _N2, _C1, _C2 = 4, 8192, 2048, 64, 128
_NP = _B * _N1
_TILE = 512
_NT = _N1 // _TILE
_MT = 2048
_NW = 32
_PPW = _NP // _NW
_CH = 64
_TPW = _PPW // _TILE
_HPT = _TILE // _CH


def _knn_body(x1_ref, x2t_ref, i0, w0):
    b = pl.program_id(0)
    x1 = x1_ref[0]
    x2 = x2t_ref[0]
    c0, c1, c2 = x1[:, 0:1], x1[:, 1:2], x1[:, 2:3]
    r0, r1, r2 = x2[0:1, :], x2[1:2, :], x2[2:3, :]
    dot2 = lax.dot_general(x1 * -2.0, x2, (((1,), (0,)), ((), ())),
                           preferred_element_type=jnp.float32)
    x1sq = (c0 * c0 + c1 * c1) + c2 * c2
    x2sq = (r0 * r0 + r1 * r1) + r2 * r2
    d = dot2 + x1sq + x2sq
    iota = lax.broadcasted_iota(jnp.int32, (_TILE, _N2), 1)
    idxs, dists = [], []
    for k in range(3):
        m = jnp.min(d, axis=1, keepdims=True)
        am = jnp.min(jnp.where(d == m, iota, _N2), axis=1, keepdims=True)
        dists.append(m)
        idxs.append(am)
        if k < 2:
            d = jnp.where(iota == am, jnp.float32(np.inf), d)
    ws = [1.0 / jnp.maximum(x, 1e-10) for x in dists]
    norm = jnp.maximum((ws[0] + ws[1]) + ws[2], 1e-8)
    base = b * _N2
    zi = jnp.zeros((_TILE, 1), jnp.int32)
    zf = jnp.zeros((_TILE, 1), jnp.float32)
    idx_mat = jnp.concatenate(
        [idxs[0] + base, idxs[1] + base, idxs[2] + base, zi, zi, zi, zi, zi],
        axis=1)
    w_mat = jnp.concatenate(
        [ws[0] / norm, ws[1] / norm, ws[2] / norm, zf, zf, zf, zf, zf],
        axis=1)
    i0[0, 0] = jnp.transpose(idx_mat, (1, 0))
    w0[0, 0] = jnp.transpose(w_mat, (1, 0))


def _knn(xyz1, xyz2t):
    out_spec = pl.BlockSpec((1, 1, 8, _TILE), lambda b, t: (b, t, 0, 0))
    shapes = [jax.ShapeDtypeStruct((_B, _NT, 8, _TILE), jnp.int32),
              jax.ShapeDtypeStruct((_B, _NT, 8, _TILE), jnp.float32)]
    return pl.pallas_call(
        _knn_body,
        grid=(_B, _NT),
        in_specs=[pl.BlockSpec((1, _TILE, 3), lambda b, t: (b, t, 0)),
                  pl.BlockSpec((1, 3, _N2), lambda b, t: (b, 0, 0))],
        out_specs=[out_spec] * 2,
        out_shape=shapes,
    )(xyz1, xyz2t)


def _sc_interp(idx, wgt, table):
    mesh = plsc.VectorSubcoreMesh(core_axis_name="c", subcore_axis_name="s")

    @functools.partial(
        pl.kernel,
        out_type=jax.ShapeDtypeStruct((_NP, _C2), jnp.float32),
        mesh=mesh,
        scratch_types=[
            pltpu.VMEM((4 * _TPW, _TILE), jnp.int32),
            pltpu.VMEM((4 * _TPW, _TILE), jnp.float32),
            pltpu.VMEM((2, 3, _CH, _C2), jnp.float32),
            pltpu.VMEM((_CH, _C2), jnp.float32),
            pltpu.SemaphoreType.DMA,
            pltpu.SemaphoreType.DMA,
        ],
    )
    def kfn(ih, wh, th, outh, iv, wv, rv, ov, sem0, sem1):
        wid = lax.axis_index("c") * 16 + lax.axis_index("s")
        nch = _TPW * _HPT
        sems = (sem0, sem1)

        for j in range(_TPW):
            g = wid * _TPW + j
            pltpu.sync_copy(ih.at[g // _NT, g % _NT, pl.ds(0, 4)],
                            iv.at[pl.ds(4 * j, 4)])
            pltpu.sync_copy(wh.at[g // _NT, g % _NT, pl.ds(0, 4)],
                            wv.at[pl.ds(4 * j, 4)])

        def fire(c, par):
            j = c // _HPT
            h = c % _HPT
            hs = pl.ds(h * _CH, _CH)
            for k in range(3):
                pltpu.async_copy(th.at[iv.at[4 * j + k, hs]],
                                 rv.at[par, k], sems[par])

        def drain(par):
            for k in range(3):
                pltpu.make_async_copy(th.at[pl.ds(0, _CH)],
                                      rv.at[par, k], sems[par]).wait()

        fire(0, 0)

        def outer(g2, carry):
            for par in range(2):
                c = g2 * 2 + par
                drain(par)
                fire(jnp.minimum(c + 1, nch - 1), 1 - par)
                j = c // _HPT
                h = c % _HPT
                g = wid * _TPW + j
                base = (g // _NT) * _N1 + (g % _NT) * _TILE + h * _CH

                def body(q, cc):
                    qs = pl.ds(h * _CH + q * 16, 16)
                    wv0 = wv[4 * j + 0, qs]
                    wv1 = wv[4 * j + 1, qs]
                    wv2 = wv[4 * j + 2, qs]
                    for jj in range(16):
                        p = q * 16 + jj
                        wb0 = jnp.full((16,), wv0[jj], jnp.float32)
                        wb1 = jnp.full((16,), wv1[jj], jnp.float32)
                        wb2 = jnp.full((16,), wv2[jj], jnp.float32)
                        for gg in range(_C2 // 16):
                            s = pl.ds(gg * 16, 16)
                            ov[p, s] = (wb0 * rv[par, 0, p, s]
                                        + wb1 * rv[par, 1, p, s]
                                        + wb2 * rv[par, 2, p, s])
                    return cc

                lax.fori_loop(0, _CH // 16, body, 0)
                pltpu.sync_copy(ov, outh.at[pl.ds(base, _CH)])
            return carry

        lax.fori_loop(0, nch // 2, outer, 0)
        drain(0)

    return kfn(idx, wgt, table)


def _dot(a, b):
    return jnp.dot(a, b, preferred_element_type=jnp.float32,
                   precision=lax.Precision.HIGHEST)


def _acc_stats(i, y, st_ref):
    @pl.when(i == 0)
    def _():
        st_ref[...] = jnp.zeros_like(st_ref)
    st_ref[0:1, :] = st_ref[0:1, :] + jnp.sum(y, axis=0, keepdims=True)
    st_ref[1:2, :] = st_ref[1:2, :] + jnp.sum(y * y, axis=0, keepdims=True)


def _bn(y, st_ref, g_ref, be_ref):
    mean = st_ref[0:1, :] * (1.0 / _NP)
    q = st_ref[1:2, :] * (1.0 / _NP)
    rstd = lax.rsqrt((q - mean * mean) + 1e-5)
    return jnp.maximum((y - mean) * rstd * g_ref[...] + be_ref[...], 0.0)


def _mlp0_body(p1_ref, it_ref, wa_ref, wb_ref, b0_ref, y_ref, st_ref):
    i = pl.program_id(0)
    y = _dot(p1_ref[...], wa_ref[...]) + _dot(it_ref[...], wb_ref[...])
    y = y + b0_ref[...]
    y_ref[...] = y
    _acc_stats(i, y, st_ref)


def _mlp1_body(y0_ref, st_ref, g_ref, be_ref, w_ref, b_ref, y1_ref, st1_ref):
    i = pl.program_id(0)
    h = _bn(y0_ref[...], st_ref, g_ref, be_ref)
    y = _dot(h, w_ref[...]) + b_ref[...]
    y1_ref[...] = y
    _acc_stats(i, y, st1_ref)


def _mlp2_body(y1_ref, st_ref, g_ref, be_ref, o_ref):
    o_ref[...] = _bn(y1_ref[...], st_ref, g_ref, be_ref)


def _row_spec(c):
    return pl.BlockSpec((_MT, c), lambda i: (i, 0))


def _full_spec(r, c):
    return pl.BlockSpec((r, c), lambda i: (0, 0))


def _mlp0(p1, interp, waT, wbT, b0):
    return pl.pallas_call(
        _mlp0_body,
        grid=(_NP // _MT,),
        in_specs=[_row_spec(_C1), _row_spec(_C2), _full_spec(_C1, 128),
                  _full_spec(_C2, 128), _full_spec(1, 128)],
        out_specs=[_row_spec(128), _full_spec(8, 128)],
        out_shape=[jax.ShapeDtypeStruct((_NP, 128), jnp.float32),
                   jax.ShapeDtypeStruct((8, 128), jnp.float32)],
    )(p1, interp, waT, wbT, b0)


def _mlp1(y0, st0, g0, beta0, w1T, b1):
    return pl.pallas_call(
        _mlp1_body,
        grid=(_NP // _MT,),
        in_specs=[_row_spec(128), _full_spec(8, 128), _full_spec(1, 128),
                  _full_spec(1, 128), _full_spec(128, 64), _full_spec(1, 64)],
        out_specs=[_row_spec(64), _full_spec(8, 64)],
        out_shape=[jax.ShapeDtypeStruct((_NP, 64), jnp.float32),
                   jax.ShapeDtypeStruct((8, 64), jnp.float32)],
    )(y0, st0, g0, beta0, w1T, b1)


def _mlp2(y1, st1, g1, beta1):
    return pl.pallas_call(
        _mlp2_body,
        grid=(_NP // _MT,),
        in_specs=[_row_spec(64), _full_spec(8, 64), _full_spec(1, 64),
                  _full_spec(1, 64)],
        out_specs=_row_spec(64),
        out_shape=jax.ShapeDtypeStruct((_NP, 64), jnp.float32),
    )(y1, st1, g1, beta1)


def kernel(xyz1, xyz2, points1, points2, W0, b0, g0, beta0,
           W1, b1, g1, beta1):
    xyz2t = jnp.transpose(xyz2, (0, 2, 1))
    idx, wgt = _knn(xyz1, xyz2t)
    table = points2.reshape(_B * _N2, _C2)
    interp = _sc_interp(idx, wgt, table)
    p1 = points1.reshape(_NP, _C1)
    waT = W0[:, :_C1].T
    wbT = W0[:, _C1:].T
    y0, st0 = _mlp0(p1, interp, waT, wbT, b0.reshape(1, 128))
    y1, st1 = _mlp1(y0, st0, g0.reshape(1, 128), beta0.reshape(1, 128),
                    W1.T, b1.reshape(1, 64))
    out = _mlp2(y1, st1, g1.reshape(1, 64), beta1.reshape(1, 64))
    return out.reshape(_B, _N1, _C1)


# speedup vs baseline: 33.7338x; 1.0569x over previous
import functools

import jax
import jax.numpy as jnp
import numpy as np
from jax import lax
from jax.experimental import pallas as pl
from jax.experimental.pallas import tpu as pltpu
from jax.experimental.pallas import tpu_sc as plsc

_B, _N1, _N2, _C1, _C2 = 4, 8192, 2048, 64, 128
_NP = _B * _N1
_TILE = 512
_NT = _N1 // _TILE
_MT = 2048
_NW = 32
_PPW = _NP // _NW
_CH = 64
_TPW = _PPW // _TILE
_HPT = _TILE // _CH


def _knn_body(b_off, x1_ref, x2t_ref, i0, w0):
    b = pl.program_id(0) + b_off
    x1 = x1_ref[0]
    x2 = x2t_ref[0]
    c0, c1, c2 = x1[:, 0:1], x1[:, 1:2], x1[:, 2:3]
    r0, r1, r2 = x2[0:1, :], x2[1:2, :], x2[2:3, :]
    dot2 = lax.dot_general(x1 * -2.0, x2, (((1,), (0,)), ((), ())),
                           preferred_element_type=jnp.float32)
    x1sq = (c0 * c0 + c1 * c1) + c2 * c2
    x2sq = (r0 * r0 + r1 * r1) + r2 * r2
    d = dot2 + x1sq + x2sq
    iota = lax.broadcasted_iota(jnp.int32, (_TILE, _N2), 1)
    idxs, dists = [], []
    for k in range(3):
        m = jnp.min(d, axis=1, keepdims=True)
        am = jnp.min(jnp.where(d == m, iota, _N2), axis=1, keepdims=True)
        dists.append(m)
        idxs.append(am)
        if k < 2:
            d = jnp.where(iota == am, jnp.float32(np.inf), d)
    ws = [1.0 / jnp.maximum(x, 1e-10) for x in dists]
    norm = jnp.maximum((ws[0] + ws[1]) + ws[2], 1e-8)
    base = b * _N2
    zi = jnp.zeros((_TILE, 1), jnp.int32)
    zf = jnp.zeros((_TILE, 1), jnp.float32)
    idx_mat = jnp.concatenate(
        [idxs[0] + base, idxs[1] + base, idxs[2] + base, zi, zi, zi, zi, zi],
        axis=1)
    w_mat = jnp.concatenate(
        [ws[0] / norm, ws[1] / norm, ws[2] / norm, zf, zf, zf, zf, zf],
        axis=1)
    i0[0, 0] = jnp.transpose(idx_mat, (1, 0))
    w0[0, 0] = jnp.transpose(w_mat, (1, 0))


def _knn(xyz1, xyz2t, b_off):
    bh = xyz1.shape[0]
    out_spec = pl.BlockSpec((1, 1, 8, _TILE), lambda b, t: (b, t, 0, 0))
    shapes = [jax.ShapeDtypeStruct((bh, _NT, 8, _TILE), jnp.int32),
              jax.ShapeDtypeStruct((bh, _NT, 8, _TILE), jnp.float32)]
    return pl.pallas_call(
        functools.partial(_knn_body, b_off),
        grid=(bh, _NT),
        in_specs=[pl.BlockSpec((1, _TILE, 3), lambda b, t: (b, t, 0)),
                  pl.BlockSpec((1, 3, _N2), lambda b, t: (b, 0, 0))],
        out_specs=[out_spec] * 2,
        out_shape=shapes,
    )(xyz1, xyz2t)


def _sc_interp(idx, wgt, table):
    bh = idx.shape[0]
    nph = bh * _N1
    tpw = nph // (_NW * _TILE)
    mesh = plsc.VectorSubcoreMesh(core_axis_name="c", subcore_axis_name="s")

    @functools.partial(
        pl.kernel,
        out_type=jax.ShapeDtypeStruct((nph, _C2), jnp.float32),
        mesh=mesh,
        scratch_types=[
            pltpu.VMEM((4 * tpw, _TILE), jnp.int32),
            pltpu.VMEM((4 * tpw, _TILE), jnp.float32),
            pltpu.VMEM((2, 3, _CH, _C2), jnp.float32),
            pltpu.VMEM((_CH, _C2), jnp.float32),
            pltpu.SemaphoreType.DMA,
            pltpu.SemaphoreType.DMA,
        ],
    )
    def kfn(ih, wh, th, outh, iv, wv, rv, ov, sem0, sem1):
        wid = lax.axis_index("c") * 16 + lax.axis_index("s")
        nch = tpw * _HPT
        sems = (sem0, sem1)

        for j in range(tpw):
            g = wid * tpw + j
            pltpu.sync_copy(ih.at[g // _NT, g % _NT, pl.ds(0, 4)],
                            iv.at[pl.ds(4 * j, 4)])
            pltpu.sync_copy(wh.at[g // _NT, g % _NT, pl.ds(0, 4)],
                            wv.at[pl.ds(4 * j, 4)])

        def fire(c, par):
            j = c // _HPT
            h = c % _HPT
            hs = pl.ds(h * _CH, _CH)
            for k in range(3):
                pltpu.async_copy(th.at[iv.at[4 * j + k, hs]],
                                 rv.at[par, k], sems[par])

        def drain(par):
            for k in range(3):
                pltpu.make_async_copy(th.at[pl.ds(0, _CH)],
                                      rv.at[par, k], sems[par]).wait()

        fire(0, 0)

        def outer(g2, carry):
            for par in range(2):
                c = g2 * 2 + par
                drain(par)
                fire(jnp.minimum(c + 1, nch - 1), 1 - par)
                j = c // _HPT
                h = c % _HPT
                g = wid * tpw + j
                base = (g // _NT) * _N1 + (g % _NT) * _TILE + h * _CH

                def body(q, cc):
                    qs = pl.ds(h * _CH + q * 16, 16)
                    wv0 = wv[4 * j + 0, qs]
                    wv1 = wv[4 * j + 1, qs]
                    wv2 = wv[4 * j + 2, qs]
                    for jj in range(16):
                        p = q * 16 + jj
                        wb0 = jnp.full((16,), wv0[jj], jnp.float32)
                        wb1 = jnp.full((16,), wv1[jj], jnp.float32)
                        wb2 = jnp.full((16,), wv2[jj], jnp.float32)
                        for gg in range(_C2 // 16):
                            s = pl.ds(gg * 16, 16)
                            ov[p, s] = (wb0 * rv[par, 0, p, s]
                                        + wb1 * rv[par, 1, p, s]
                                        + wb2 * rv[par, 2, p, s])
                    return cc

                lax.fori_loop(0, _CH // 16, body, 0)
                pltpu.sync_copy(ov, outh.at[pl.ds(base, _CH)])
            return carry

        lax.fori_loop(0, nch // 2, outer, 0)
        drain(0)

    return kfn(idx, wgt, table)


def _dot(a, b):
    return jnp.dot(a, b, preferred_element_type=jnp.float32,
                   precision=lax.Precision.HIGHEST)


def _acc_stats(i, y, st_ref):
    @pl.when(i == 0)
    def _():
        st_ref[...] = jnp.zeros_like(st_ref)
    st_ref[0:1, :] = st_ref[0:1, :] + jnp.sum(y, axis=0, keepdims=True)
    st_ref[1:2, :] = st_ref[1:2, :] + jnp.sum(y * y, axis=0, keepdims=True)


def _bn(y, sta_ref, stb_ref, g_ref, be_ref):
    s = sta_ref[...] + stb_ref[...]
    mean = s[0:1, :] * (1.0 / _NP)
    q = s[1:2, :] * (1.0 / _NP)
    rstd = lax.rsqrt((q - mean * mean) + 1e-5)
    return jnp.maximum((y - mean) * rstd * g_ref[...] + be_ref[...], 0.0)


def _mlp0_body(p1_ref, it_ref, wa_ref, wb_ref, b0_ref, y_ref, st_ref):
    i = pl.program_id(0)
    y = _dot(p1_ref[...], wa_ref[...]) + _dot(it_ref[...], wb_ref[...])
    y = y + b0_ref[...]
    y_ref[...] = y
    _acc_stats(i, y, st_ref)


def _mlp1_body(y0_ref, sta_ref, stb_ref, g_ref, be_ref, w_ref, b_ref,
               y1_ref, st1_ref):
    i = pl.program_id(0)
    h = _bn(y0_ref[...], sta_ref, stb_ref, g_ref, be_ref)
    y = _dot(h, w_ref[...]) + b_ref[...]
    y1_ref[...] = y
    _acc_stats(i, y, st1_ref)


def _mlp2_body(y1_ref, sta_ref, stb_ref, g_ref, be_ref, o_ref):
    o_ref[...] = _bn(y1_ref[...], sta_ref, stb_ref, g_ref, be_ref)


def _row_spec(c):
    return pl.BlockSpec((_MT, c), lambda i: (i, 0))


def _full_spec(r, c):
    return pl.BlockSpec((r, c), lambda i: (0, 0))


def _mlp0(p1, interp, waT, wbT, b0):
    n = p1.shape[0]
    return pl.pallas_call(
        _mlp0_body,
        grid=(n // _MT,),
        in_specs=[_row_spec(_C1), _row_spec(_C2), _full_spec(_C1, 128),
                  _full_spec(_C2, 128), _full_spec(1, 128)],
        out_specs=[_row_spec(128), _full_spec(8, 128)],
        out_shape=[jax.ShapeDtypeStruct((n, 128), jnp.float32),
                   jax.ShapeDtypeStruct((8, 128), jnp.float32)],
    )(p1, interp, waT, wbT, b0)


def _mlp1(y0, sta, stb, g0, beta0, w1T, b1):
    n = y0.shape[0]
    return pl.pallas_call(
        _mlp1_body,
        grid=(n // _MT,),
        in_specs=[_row_spec(128), _full_spec(8, 128), _full_spec(8, 128),
                  _full_spec(1, 128), _full_spec(1, 128),
                  _full_spec(128, 64), _full_spec(1, 64)],
        out_specs=[_row_spec(64), _full_spec(8, 64)],
        out_shape=[jax.ShapeDtypeStruct((n, 64), jnp.float32),
                   jax.ShapeDtypeStruct((8, 64), jnp.float32)],
    )(y0, sta, stb, g0, beta0, w1T, b1)


def _mlp2(y1, sta, stb, g1, beta1):
    n = y1.shape[0]
    return pl.pallas_call(
        _mlp2_body,
        grid=(n // _MT,),
        in_specs=[_row_spec(64), _full_spec(8, 64), _full_spec(8, 64),
                  _full_spec(1, 64), _full_spec(1, 64)],
        out_specs=_row_spec(64),
        out_shape=jax.ShapeDtypeStruct((n, 64), jnp.float32),
    )(y1, sta, stb, g1, beta1)


def kernel(xyz1, xyz2, points1, points2, W0, b0, g0, beta0,
           W1, b1, g1, beta1):
    bh = _B // 2
    xyz2t = jnp.transpose(xyz2, (0, 2, 1))
    table = points2.reshape(_B * _N2, _C2)
    waT = W0[:, :_C1].T
    wbT = W0[:, _C1:].T
    b0r = b0.reshape(1, 128)
    idxA, wgtA = _knn(xyz1[:bh], xyz2t[:bh], 0)
    idxB, wgtB = _knn(xyz1[bh:], xyz2t[bh:], bh)
    interpA = _sc_interp(idxA, wgtA, table)
    interpB = _sc_interp(idxB, wgtB, table)
    nph = bh * _N1
    p1 = points1.reshape(_NP, _C1)
    y0A, stA = _mlp0(p1[:nph], interpA, waT, wbT, b0r)
    y0B, stB = _mlp0(p1[nph:], interpB, waT, wbT, b0r)
    g0r, be0r = g0.reshape(1, 128), beta0.reshape(1, 128)
    w1T, b1r = W1.T, b1.reshape(1, 64)
    y1A, st1A = _mlp1(y0A, stA, stB, g0r, be0r, w1T, b1r)
    y1B, st1B = _mlp1(y0B, stA, stB, g0r, be0r, w1T, b1r)
    g1r, be1r = g1.reshape(1, 64), beta1.reshape(1, 64)
    outA = _mlp2(y1A, st1A, st1B, g1r, be1r)
    outB = _mlp2(y1B, st1A, st1B, g1r, be1r)
    return jnp.concatenate([outA, outB], axis=0).reshape(_B, _N1, _C1)
